# baseline (speedup 1.0000x reference)
import jax
import jax.numpy as jnp
from jax import lax
from jax.experimental import pallas as pl
from jax.experimental.pallas import tpu as pltpu

_LINK_T = 0.2
_IOU_T = 0.2
_NEG = -1e30
_HAS_T = -1e29
_BIGI = 2 ** 30


def _seq_nms_body(x1, y1, x2, y2, x1t, y1t, x2t, y2t, cls, clst, scores,
                  out, gt, ms, alive):
    F, N = scores.shape

    X1, Y1, X2, Y2 = x1[:], y1[:], x2[:], y2[:]
    X1T, Y1T, X2T, Y2T = x1t[:], y1t[:], x2t[:], y2t[:]
    CLS, CLST = cls[:], clst[:]
    AREA = (X2 - X1) * (Y2 - Y1)
    AREAT = (X2T - X1T) * (Y2T - Y1T)

    lane_i = lax.broadcasted_iota(jnp.int32, (1, N), 1)
    frame_col = lax.broadcasted_iota(jnp.int32, (F, 1), 0)
    lane3 = lax.broadcasted_iota(jnp.int32, (1, N, N), 2)

    for f in range(F - 1):
        a_x1, a_y1 = X1[f:f + 1, :], Y1[f:f + 1, :]
        a_x2, a_y2 = X2[f:f + 1, :], Y2[f:f + 1, :]
        b_x1, b_y1 = X1T[:, f + 1:f + 2], Y1T[:, f + 1:f + 2]
        b_x2, b_y2 = X2T[:, f + 1:f + 2], Y2T[:, f + 1:f + 2]
        ltx = jnp.maximum(a_x1, b_x1)
        lty = jnp.maximum(a_y1, b_y1)
        rbx = jnp.minimum(a_x2, b_x2)
        rby = jnp.minimum(a_y2, b_y2)
        inter = jnp.maximum(rbx - ltx, 0.0) * jnp.maximum(rby - lty, 0.0)
        union = AREA[f:f + 1, :] + AREAT[:, f + 1:f + 2] - inter
        iou = inter / jnp.maximum(union, 1e-9)
        edge = (iou >= _LINK_T) & (CLS[f:f + 1, :] == CLST[:, f + 1:f + 2])
        gt[f, :, :] = jnp.where(edge, 0.0, _NEG)

    out[:] = scores[:]
    alive[:] = jnp.full((F, N), 1.0, jnp.float32)

    def dp_frame(f):
        nrow_next = ms[pl.ds(f + 1, 1), :]
        alive_next = alive[pl.ds(f + 1, 1), :] > 0.5
        c = jnp.transpose(jnp.where(alive_next, nrow_next, _NEG))
        masked = c + gt[f, :, :]
        best_v = jnp.max(masked, axis=0, keepdims=True)
        a_row = alive[pl.ds(f, 1), :] > 0.5
        has = (best_v > _HAS_T) & a_row
        s_f = out[pl.ds(f, 1), :]
        row = s_f + jnp.where(has, best_v, 0.0)
        ms[pl.ds(f, 1), :] = row

    def iteration(carry):
        it, _, fe_prev = carry

        ms[pl.ds(F - 1, 1), :] = out[pl.ds(F - 1, 1), :]
        for f in range(F - 2, -1, -1):
            lax.cond(f <= fe_prev, lambda f=f: dp_frame(f), lambda: None)

        M = ms[:]
        gm = jnp.max(M)
        rowmax = jnp.max(M, axis=1, keepdims=True)
        f0 = jnp.min(jnp.where(rowmax == gm, frame_col, _BIGI))
        row0 = ms[pl.ds(f0, 1), :]
        i0 = jnp.min(jnp.where(row0 == gm, lane_i, _BIGI))

        def tstep(tc):
            fc, ic, in_seq, bidx, length, _ = tc
            fn = jnp.minimum(fc + 1, F - 1)
            cand = jnp.where(alive[pl.ds(fn, 1), :] > 0.5,
                             ms[pl.ds(fn, 1), :], _NEG)
            gsl = gt[pl.ds(jnp.minimum(fc, F - 2), 1), :, :]
            gcol = jnp.max(jnp.where(lane3 == ic, gsl, _NEG), axis=2)
            cand = cand + gcol
            best_v = jnp.max(cand)
            best_j = jnp.min(jnp.where(cand == best_v, lane_i, _BIGI))
            alv_i = jnp.sum(jnp.where(lane_i == ic,
                                      alive[pl.ds(fc, 1), :], 0.0)) > 0.5
            cond = (best_v > _HAS_T) & alv_i & (fc < F - 1)
            nf = jnp.where(cond, fc + 1, fc)
            nb = jnp.where(cond, best_j, ic)
            in_seq = jnp.where(frame_col == nf, 1.0, in_seq)
            bidx = jnp.where(frame_col == nf, nb, bidx)
            return (nf, nb, in_seq, bidx,
                    length + cond.astype(jnp.int32), cond)

        in_seq0 = jnp.where(frame_col == f0, 1.0,
                            jnp.zeros((F, 1), jnp.float32))
        bidx0 = jnp.where(frame_col == f0, i0,
                          jnp.zeros((F, 1), jnp.int32))
        (fe, _, in_seq, bidx, length, _) = lax.while_loop(
            lambda tc: tc[5], tstep,
            (f0, i0, in_seq0, bidx0, jnp.int32(1), jnp.bool_(True)))

        active = length > 1
        avg = gm / length.astype(jnp.float32)
        onehot = lane_i == bidx
        ohf = jnp.where(onehot, 1.0, 0.0)
        sx1 = jnp.sum(X1 * ohf, axis=1, keepdims=True)
        sy1 = jnp.sum(Y1 * ohf, axis=1, keepdims=True)
        sx2 = jnp.sum(X2 * ohf, axis=1, keepdims=True)
        sy2 = jnp.sum(Y2 * ohf, axis=1, keepdims=True)
        sarea = (sx2 - sx1) * (sy2 - sy1)
        ltx = jnp.maximum(sx1, X1)
        lty = jnp.maximum(sy1, Y1)
        rbx = jnp.minimum(sx2, X2)
        rby = jnp.minimum(sy2, Y2)
        inter = jnp.maximum(rbx - ltx, 0.0) * jnp.maximum(rby - lty, 0.0)
        union = sarea + AREA - inter
        iou = inter / jnp.maximum(union, 1e-9)
        insb = in_seq > 0.5
        dmask = insb & (iou >= _IOU_T) & active
        seq_sel = insb & onehot
        sc = out[:]
        sc = jnp.where(seq_sel & active, avg, sc)
        sc = jnp.where(dmask & jnp.logical_not(seq_sel), 0.0, sc)
        out[:] = sc
        alive[:] = jnp.where(dmask, 0.0, alive[:])
        return (it + 1, length <= 1, fe)

    lax.while_loop(
        lambda carr: (carr[0] < F * N) & jnp.logical_not(carr[1]),
        iteration, (jnp.int32(0), jnp.bool_(False), jnp.int32(F - 1)))


def kernel(boxes, scores, classes):
    b = jnp.asarray(boxes, jnp.float32)
    s = jnp.asarray(scores, jnp.float32)
    c = jnp.asarray(classes).astype(jnp.float32)
    F, N = s.shape
    x1, y1, x2, y2 = b[..., 0], b[..., 1], b[..., 2], b[..., 3]
    out = pl.pallas_call(
        _seq_nms_body,
        out_shape=jax.ShapeDtypeStruct((F, N), jnp.float32),
        scratch_shapes=[
            pltpu.VMEM((F - 1, N, N), jnp.float32),
            pltpu.VMEM((F, N), jnp.float32),
            pltpu.VMEM((F, N), jnp.float32),
        ],
    )(x1, y1, x2, y2, x1.T, y1.T, x2.T, y2.T, c, c.T, s)
    return out


# speedup vs baseline: 1.1312x; 1.1312x over previous
import jax
import jax.numpy as jnp
from jax import lax
from jax.experimental import pallas as pl
from jax.experimental.pallas import tpu as pltpu

_LINK_T = 0.2
_IOU_T = 0.2
_NEG = -1e30
_HAS_T = -1e29
_BIGI = 2 ** 30


def _seq_nms_body(x1, y1, x2, y2, x1t, y1t, x2t, y2t, cls, clst, scores,
                  out, gt, gn, ms, alive):
    F, N = scores.shape

    X1, Y1, X2, Y2 = x1[:], y1[:], x2[:], y2[:]
    X1T, Y1T, X2T, Y2T = x1t[:], y1t[:], x2t[:], y2t[:]
    CLS, CLST = cls[:], clst[:]
    AREA = (X2 - X1) * (Y2 - Y1)
    AREAT = (X2T - X1T) * (Y2T - Y1T)

    lane_i = lax.broadcasted_iota(jnp.int32, (1, N), 1)
    frame_col = lax.broadcasted_iota(jnp.int32, (F, 1), 0)

    for f in range(F - 1):
        a_x1, a_y1 = X1[f:f + 1, :], Y1[f:f + 1, :]
        a_x2, a_y2 = X2[f:f + 1, :], Y2[f:f + 1, :]
        b_x1, b_y1 = X1T[:, f + 1:f + 2], Y1T[:, f + 1:f + 2]
        b_x2, b_y2 = X2T[:, f + 1:f + 2], Y2T[:, f + 1:f + 2]
        ltx = jnp.maximum(a_x1, b_x1)
        lty = jnp.maximum(a_y1, b_y1)
        rbx = jnp.minimum(a_x2, b_x2)
        rby = jnp.minimum(a_y2, b_y2)
        inter = jnp.maximum(rbx - ltx, 0.0) * jnp.maximum(rby - lty, 0.0)
        union = AREA[f:f + 1, :] + AREAT[:, f + 1:f + 2] - inter
        iou = inter / jnp.maximum(union, 1e-9)
        edge = (iou >= _LINK_T) & (CLS[f:f + 1, :] == CLST[:, f + 1:f + 2])
        gt[f, :, :] = jnp.where(edge, 0.0, _NEG)
        ltx2 = jnp.maximum(X1T[:, f:f + 1], X1[f + 1:f + 2, :])
        lty2 = jnp.maximum(Y1T[:, f:f + 1], Y1[f + 1:f + 2, :])
        rbx2 = jnp.minimum(X2T[:, f:f + 1], X2[f + 1:f + 2, :])
        rby2 = jnp.minimum(Y2T[:, f:f + 1], Y2[f + 1:f + 2, :])
        inter2 = (jnp.maximum(rbx2 - ltx2, 0.0)
                  * jnp.maximum(rby2 - lty2, 0.0))
        union2 = AREAT[:, f:f + 1] + AREA[f + 1:f + 2, :] - inter2
        iou2 = inter2 / jnp.maximum(union2, 1e-9)
        edge2 = (iou2 >= _LINK_T) & (CLST[:, f:f + 1] == CLS[f + 1:f + 2, :])
        gn[f, :, :] = jnp.where(edge2, 0.0, _NEG)

    out[:] = scores[:]
    alive[:] = jnp.full((F, N), 1.0, jnp.float32)

    def dp_frame(f):
        nrow_next = ms[pl.ds(f + 1, 1), :]
        alive_next = alive[pl.ds(f + 1, 1), :] > 0.5
        c = jnp.transpose(jnp.where(alive_next, nrow_next, _NEG))
        masked = c + gt[f, :, :]
        best_v = jnp.max(masked, axis=0, keepdims=True)
        a_row = alive[pl.ds(f, 1), :] > 0.5
        has = (best_v > _HAS_T) & a_row
        s_f = out[pl.ds(f, 1), :]
        row = s_f + jnp.where(has, best_v, 0.0)
        ms[pl.ds(f, 1), :] = row

    def iteration(carry):
        it, _, fe_prev = carry

        ms[pl.ds(F - 1, 1), :] = out[pl.ds(F - 1, 1), :]
        for f in range(F - 2, -1, -1):
            lax.cond(f <= fe_prev, lambda f=f: dp_frame(f), lambda: None)

        M = ms[:]
        gm = jnp.max(M)
        rowmax = jnp.max(M, axis=1, keepdims=True)
        f0 = jnp.min(jnp.where(rowmax == gm, frame_col, _BIGI))
        row0 = ms[pl.ds(f0, 1), :]
        i0 = jnp.min(jnp.where(row0 == gm, lane_i, _BIGI))

        def tstep(tc):
            fc, ic, in_seq, bidx, length, _ = tc
            fn = jnp.minimum(fc + 1, F - 1)
            cand = jnp.where(alive[pl.ds(fn, 1), :] > 0.5,
                             ms[pl.ds(fn, 1), :], _NEG)
            grow = gn[pl.ds(jnp.minimum(fc, F - 2), 1),
                      pl.ds(ic, 1), :]
            cand = cand + grow[0]
            best_v = jnp.max(cand)
            best_j = jnp.min(jnp.where(cand == best_v, lane_i, _BIGI))
            alv_i = jnp.sum(jnp.where(lane_i == ic,
                                      alive[pl.ds(fc, 1), :], 0.0)) > 0.5
            cond = (best_v > _HAS_T) & alv_i & (fc < F - 1)
            nf = jnp.where(cond, fc + 1, fc)
            nb = jnp.where(cond, best_j, ic)
            in_seq = jnp.where(frame_col == nf, 1.0, in_seq)
            bidx = jnp.where(frame_col == nf, nb, bidx)
            return (nf, nb, in_seq, bidx,
                    length + cond.astype(jnp.int32), cond)

        in_seq0 = jnp.where(frame_col == f0, 1.0,
                            jnp.zeros((F, 1), jnp.float32))
        bidx0 = jnp.where(frame_col == f0, i0,
                          jnp.zeros((F, 1), jnp.int32))
        (fe, _, in_seq, bidx, length, _) = lax.while_loop(
            lambda tc: tc[5], tstep,
            (f0, i0, in_seq0, bidx0, jnp.int32(1), jnp.bool_(True)))

        active = length > 1
        avg = gm / length.astype(jnp.float32)
        onehot = lane_i == bidx
        ohf = jnp.where(onehot, 1.0, 0.0)
        sx1 = jnp.sum(X1 * ohf, axis=1, keepdims=True)
        sy1 = jnp.sum(Y1 * ohf, axis=1, keepdims=True)
        sx2 = jnp.sum(X2 * ohf, axis=1, keepdims=True)
        sy2 = jnp.sum(Y2 * ohf, axis=1, keepdims=True)
        sarea = (sx2 - sx1) * (sy2 - sy1)
        ltx = jnp.maximum(sx1, X1)
        lty = jnp.maximum(sy1, Y1)
        rbx = jnp.minimum(sx2, X2)
        rby = jnp.minimum(sy2, Y2)
        inter = jnp.maximum(rbx - ltx, 0.0) * jnp.maximum(rby - lty, 0.0)
        union = sarea + AREA - inter
        iou = inter / jnp.maximum(union, 1e-9)
        insb = in_seq > 0.5
        dmask = insb & (iou >= _IOU_T) & active
        seq_sel = insb & onehot
        sc = out[:]
        sc = jnp.where(seq_sel & active, avg, sc)
        sc = jnp.where(dmask & jnp.logical_not(seq_sel), 0.0, sc)
        out[:] = sc
        alive[:] = jnp.where(dmask, 0.0, alive[:])
        return (it + 1, length <= 1, fe)

    lax.while_loop(
        lambda carr: (carr[0] < F * N) & jnp.logical_not(carr[1]),
        iteration, (jnp.int32(0), jnp.bool_(False), jnp.int32(F - 1)))


def kernel(boxes, scores, classes):
    b = jnp.asarray(boxes, jnp.float32)
    s = jnp.asarray(scores, jnp.float32)
    c = jnp.asarray(classes).astype(jnp.float32)
    F, N = s.shape
    x1, y1, x2, y2 = b[..., 0], b[..., 1], b[..., 2], b[..., 3]
    out = pl.pallas_call(
        _seq_nms_body,
        out_shape=jax.ShapeDtypeStruct((F, N), jnp.float32),
        scratch_shapes=[
            pltpu.VMEM((F - 1, N, N), jnp.float32),
            pltpu.VMEM((F - 1, N, N), jnp.float32),
            pltpu.VMEM((F, N), jnp.float32),
            pltpu.VMEM((F, N), jnp.float32),
        ],
    )(x1, y1, x2, y2, x1.T, y1.T, x2.T, y2.T, c, c.T, s)
    return out


# speedup vs baseline: 1.8687x; 1.6520x over previous
import jax
import jax.numpy as jnp
from jax import lax
from jax.experimental import pallas as pl
from jax.experimental.pallas import tpu as pltpu

_LINK_T = 0.2
_IOU_T = 0.2
_NEG = -1e30
_HAS_T = -1e29
_BIGI = 2 ** 30


def _seq_nms_body(x1, y1, x2, y2, x1t, y1t, x2t, y2t, cls, clst, scores,
                  out, gt, gn, ms, alive):
    F, N = scores.shape

    X1, Y1, X2, Y2 = x1[:], y1[:], x2[:], y2[:]
    X1T, Y1T, X2T, Y2T = x1t[:], y1t[:], x2t[:], y2t[:]
    CLS, CLST = cls[:], clst[:]
    AREA = (X2 - X1) * (Y2 - Y1)
    AREAT = (X2T - X1T) * (Y2T - Y1T)

    lane_i = lax.broadcasted_iota(jnp.int32, (1, N), 1)
    frame_col = lax.broadcasted_iota(jnp.int32, (F, 1), 0)

    for f in range(F - 1):
        a_x1, a_y1 = X1[f:f + 1, :], Y1[f:f + 1, :]
        a_x2, a_y2 = X2[f:f + 1, :], Y2[f:f + 1, :]
        b_x1, b_y1 = X1T[:, f + 1:f + 2], Y1T[:, f + 1:f + 2]
        b_x2, b_y2 = X2T[:, f + 1:f + 2], Y2T[:, f + 1:f + 2]
        ltx = jnp.maximum(a_x1, b_x1)
        lty = jnp.maximum(a_y1, b_y1)
        rbx = jnp.minimum(a_x2, b_x2)
        rby = jnp.minimum(a_y2, b_y2)
        inter = jnp.maximum(rbx - ltx, 0.0) * jnp.maximum(rby - lty, 0.0)
        union = AREA[f:f + 1, :] + AREAT[:, f + 1:f + 2] - inter
        iou = inter / jnp.maximum(union, 1e-9)
        edge = (iou >= _LINK_T) & (CLS[f:f + 1, :] == CLST[:, f + 1:f + 2])
        gt[f, :, :] = jnp.where(edge, 0.0, _NEG)
        ltx2 = jnp.maximum(X1T[:, f:f + 1], X1[f + 1:f + 2, :])
        lty2 = jnp.maximum(Y1T[:, f:f + 1], Y1[f + 1:f + 2, :])
        rbx2 = jnp.minimum(X2T[:, f:f + 1], X2[f + 1:f + 2, :])
        rby2 = jnp.minimum(Y2T[:, f:f + 1], Y2[f + 1:f + 2, :])
        inter2 = (jnp.maximum(rbx2 - ltx2, 0.0)
                  * jnp.maximum(rby2 - lty2, 0.0))
        union2 = AREAT[:, f:f + 1] + AREA[f + 1:f + 2, :] - inter2
        iou2 = inter2 / jnp.maximum(union2, 1e-9)
        edge2 = (iou2 >= _LINK_T) & (CLST[:, f:f + 1] == CLS[f + 1:f + 2, :])
        gn[f, :, :] = jnp.where(edge2, 0.0, _NEG)

    out[:] = scores[:]
    alive[:] = jnp.full((F, N), 1.0, jnp.float32)

    def iteration(carry):
        it, _, fe_prev, f0_prev = carry

        ms[pl.ds(F - 1, 1), :] = out[pl.ds(F - 1, 1), :]

        def dp_body(st):
            f, _ = st
            nrow_next = ms[pl.ds(f + 1, 1), :]
            alive_next = alive[pl.ds(f + 1, 1), :] > 0.5
            c = jnp.transpose(jnp.where(alive_next, nrow_next, _NEG))
            masked = c + gt[pl.ds(f, 1), :, :][0]
            best_v = jnp.max(masked, axis=0, keepdims=True)
            a_row = alive[pl.ds(f, 1), :] > 0.5
            has = (best_v > _HAS_T) & a_row
            row = out[pl.ds(f, 1), :] + jnp.where(has, best_v, 0.0)
            old = ms[pl.ds(f, 1), :]
            ms[pl.ds(f, 1), :] = row
            settled = (f < f0_prev) & jnp.all(old == row)
            return (f - 1, settled)

        lax.while_loop(lambda st: (st[0] >= 0) & jnp.logical_not(st[1]),
                       dp_body,
                       (jnp.minimum(fe_prev, F - 2), jnp.bool_(False)))

        M = ms[:]
        gm = jnp.max(M)
        rowmax = jnp.max(M, axis=1, keepdims=True)
        f0 = jnp.min(jnp.where(rowmax == gm, frame_col, _BIGI))
        row0 = ms[pl.ds(f0, 1), :]
        i0 = jnp.min(jnp.where(row0 == gm, lane_i, _BIGI))

        def tstep(tc):
            fc, ic, in_seq, bidx, length, _ = tc
            fn = jnp.minimum(fc + 1, F - 1)
            cand = jnp.where(alive[pl.ds(fn, 1), :] > 0.5,
                             ms[pl.ds(fn, 1), :], _NEG)
            grow = gn[pl.ds(jnp.minimum(fc, F - 2), 1),
                      pl.ds(ic, 1), :]
            cand = cand + grow[0]
            best_v = jnp.max(cand)
            best_j = jnp.min(jnp.where(cand == best_v, lane_i, _BIGI))
            alv_i = jnp.sum(jnp.where(lane_i == ic,
                                      alive[pl.ds(fc, 1), :], 0.0)) > 0.5
            cond = (best_v > _HAS_T) & alv_i & (fc < F - 1)
            nf = jnp.where(cond, fc + 1, fc)
            nb = jnp.where(cond, best_j, ic)
            in_seq = jnp.where(frame_col == nf, 1.0, in_seq)
            bidx = jnp.where(frame_col == nf, nb, bidx)
            return (nf, nb, in_seq, bidx,
                    length + cond.astype(jnp.int32), cond)

        in_seq0 = jnp.where(frame_col == f0, 1.0,
                            jnp.zeros((F, 1), jnp.float32))
        bidx0 = jnp.where(frame_col == f0, i0,
                          jnp.zeros((F, 1), jnp.int32))
        (fe, _, in_seq, bidx, length, _) = lax.while_loop(
            lambda tc: tc[5], tstep,
            (f0, i0, in_seq0, bidx0, jnp.int32(1), jnp.bool_(True)))

        active = length > 1
        avg = gm / length.astype(jnp.float32)
        onehot = lane_i == bidx
        ohf = jnp.where(onehot, 1.0, 0.0)
        sx1 = jnp.sum(X1 * ohf, axis=1, keepdims=True)
        sy1 = jnp.sum(Y1 * ohf, axis=1, keepdims=True)
        sx2 = jnp.sum(X2 * ohf, axis=1, keepdims=True)
        sy2 = jnp.sum(Y2 * ohf, axis=1, keepdims=True)
        sarea = (sx2 - sx1) * (sy2 - sy1)
        ltx = jnp.maximum(sx1, X1)
        lty = jnp.maximum(sy1, Y1)
        rbx = jnp.minimum(sx2, X2)
        rby = jnp.minimum(sy2, Y2)
        inter = jnp.maximum(rbx - ltx, 0.0) * jnp.maximum(rby - lty, 0.0)
        union = sarea + AREA - inter
        iou = inter / jnp.maximum(union, 1e-9)
        insb = in_seq > 0.5
        dmask = insb & (iou >= _IOU_T) & active
        seq_sel = insb & onehot
        sc = out[:]
        sc = jnp.where(seq_sel & active, avg, sc)
        sc = jnp.where(dmask & jnp.logical_not(seq_sel), 0.0, sc)
        out[:] = sc
        alive[:] = jnp.where(dmask, 0.0, alive[:])
        return (it + 1, length <= 1, fe, f0)

    lax.while_loop(
        lambda carr: (carr[0] < F * N) & jnp.logical_not(carr[1]),
        iteration,
        (jnp.int32(0), jnp.bool_(False), jnp.int32(F - 1), jnp.int32(0)))


def kernel(boxes, scores, classes):
    b = jnp.asarray(boxes, jnp.float32)
    s = jnp.asarray(scores, jnp.float32)
    c = jnp.asarray(classes).astype(jnp.float32)
    F, N = s.shape
    x1, y1, x2, y2 = b[..., 0], b[..., 1], b[..., 2], b[..., 3]
    out = pl.pallas_call(
        _seq_nms_body,
        out_shape=jax.ShapeDtypeStruct((F, N), jnp.float32),
        scratch_shapes=[
            pltpu.VMEM((F - 1, N, N), jnp.float32),
            pltpu.VMEM((F - 1, N, N), jnp.float32),
            pltpu.VMEM((F, N), jnp.float32),
            pltpu.VMEM((F, N), jnp.float32),
        ],
    )(x1, y1, x2, y2, x1.T, y1.T, x2.T, y2.T, c, c.T, s)
    return out
